# baseline (speedup 1.0000x reference)
import functools

import jax
import jax.numpy as jnp
from jax import lax
from jax.experimental import pallas as pl
from jax.experimental.pallas import tpu as pltpu
from jax.experimental.pallas import tpu_sc as plsc

N = 10000
E = 320000
D = 128

NC = 2
NS = 16
NW = NC * NS
EPT = E // NW
K = 128
KG = 64
NBUF = 4
CHG = 160
CH = 80
N_PAD = 10112
RPT = N_PAD // NS
PAD_DST = N_PAD - 1

_mesh = functools.partial(
    plsc.VectorSubcoreMesh, core_axis_name="c", subcore_axis_name="s")


def _zero_rows(buf, rows, cols):
    zv = jnp.zeros((16,), jnp.float32)
    per_row = cols // 16

    def body(i, _):
        buf[i // per_row, pl.ds((i % per_row) * 16, 16)] = zv
        return 0

    lax.fori_loop(0, rows * per_row, body, 0, unroll=8)


def _fill_src(packed_v, ch, idx, k=K):
    for j in range(k // 16):
        w = packed_v[(ch * k + 16 * j) // K, pl.ds((ch * k + 16 * j) % K, 16)]
        idx[pl.ds(16 * j, 16)] = jnp.bitwise_and(w, 0xFFFF)


def _fill_dst(packed_v, ch, idx, k=K):
    for j in range(k // 16):
        w = packed_v[(ch * k + 16 * j) // K, pl.ds((ch * k + 16 * j) % K, 16)]
        idx[pl.ds(16 * j, 16)] = lax.shift_right_logical(w, 16)


@functools.partial(
    pl.kernel,
    out_type=jax.ShapeDtypeStruct((NC, N_PAD, D), jnp.float32),
    mesh=_mesh(),
    scratch_types=[
        pltpu.VMEM((CH, K), jnp.int32),
        pltpu.VMEM((K, D), jnp.float32),
        pltpu.VMEM((K,), jnp.int32),
        pltpu.SemaphoreType.DMA,
        pltpu.VMEM_SHARED((N_PAD, D), jnp.float32),
    ],
)
def _sc_cnt(edges_hbm, cz_hbm, out_hbm, packed_v, ones_v, di, semi, acc):
    cid = lax.axis_index("c")
    sid = lax.axis_index("s")
    wid = sid * NC + cid
    pltpu.async_copy(edges_hbm.at[wid], packed_v, semi)

    pltpu.sync_copy(cz_hbm.at[0], ones_v)
    base = sid * RPT
    for t in range(RPT // K):
        pltpu.sync_copy(ones_v, acc.at[pl.ds(base + t * K, K)])
    rem = RPT % K
    if rem:
        pltpu.sync_copy(ones_v.at[pl.ds(0, rem)],
                        acc.at[pl.ds(base + (RPT // K) * K, rem)])
    pltpu.sync_copy(cz_hbm.at[1], ones_v)
    pltpu.make_async_copy(edges_hbm.at[wid], packed_v, semi).wait()
    plsc.subcore_barrier()

    def body(ch, _):
        _fill_dst(packed_v, ch, di)
        pltpu.sync_copy(ones_v, acc.at[di], add=True)
        return 0

    lax.fori_loop(0, CH, body, 0)
    plsc.subcore_barrier()
    pltpu.sync_copy(acc.at[pl.ds(base, RPT)], out_hbm.at[cid, pl.ds(base, RPT)])


@functools.partial(
    pl.kernel,
    out_type=jax.ShapeDtypeStruct((NC, N_PAD, D), jnp.float32),
    mesh=_mesh(),
    scratch_types=[
        pltpu.VMEM((CH, K), jnp.int32),
        [pltpu.VMEM((KG, D), jnp.float32) for _ in range(NBUF)],
        [pltpu.VMEM((KG,), jnp.int32) for _ in range(NBUF)],
        [pltpu.VMEM((KG,), jnp.int32) for _ in range(2)],
        [pltpu.SemaphoreType.DMA for _ in range(NBUF)],
        [pltpu.SemaphoreType.DMA for _ in range(2)],
        pltpu.SemaphoreType.DMA,
        pltpu.VMEM_SHARED((N_PAD, D), jnp.float32),
    ],
)
def _sc_agg(table_hbm, edges_hbm, out_hbm,
            packed_v, bufs, sis, dis, sems, ssems, semi, acc):
    cid = lax.axis_index("c")
    sid = lax.axis_index("s")
    wid = sid * NC + cid
    pltpu.async_copy(edges_hbm.at[wid], packed_v, semi)

    _zero_rows(bufs[0], KG, D)
    base = sid * RPT
    for t in range(RPT // KG):
        pltpu.sync_copy(bufs[0], acc.at[pl.ds(base + t * KG, KG)])
    remz = RPT % KG
    if remz:
        pltpu.sync_copy(bufs[0].at[pl.ds(0, remz)],
                        acc.at[pl.ds(base + (RPT // KG) * KG, remz)])
    pltpu.make_async_copy(edges_hbm.at[wid], packed_v, semi).wait()
    plsc.subcore_barrier()

    def gather(si, buf, sem):
        pltpu.async_copy(table_hbm.at[si], buf, sem)

    def drain_g(si, buf, sem):
        pltpu.make_async_copy(table_hbm.at[si], buf, sem).wait()

    def drain_s(buf, ssem):
        pltpu.make_async_copy(buf, acc.at[pl.ds(0, KG)], ssem).wait()

    for b in range(NBUF):
        _fill_src(packed_v, b, sis[b], KG)
        gather(sis[b], bufs[b], sems[b])

    def body(i, _):
        for b in range(NBUF):
            ch = NBUF * i + b
            drain_g(sis[b], bufs[b], sems[b])
            _fill_dst(packed_v, ch, dis[b % 2], KG)
            pltpu.async_copy(bufs[b], acc.at[dis[b % 2]], ssems[b % 2],
                             add=True)
            bp = (b - 1) % NBUF

            @pl.when((ch - 1 >= 0) & (ch - 1 + NBUF < CHG))
            def _():
                drain_s(bufs[bp], ssems[(b - 1) % 2])
                _fill_src(packed_v, ch - 1 + NBUF, sis[bp], KG)
                gather(sis[bp], bufs[bp], sems[bp])

        return 0

    lax.fori_loop(0, CHG // NBUF, body, 0)
    for t in range(NBUF):
        ch = CHG - NBUF + t
        drain_s(bufs[ch % NBUF], ssems[ch % 2])
    plsc.subcore_barrier()
    pltpu.sync_copy(acc.at[pl.ds(base, RPT)], out_hbm.at[cid, pl.ds(base, RPT)])


BM = 1000


def _dot(a, b):
    return lax.dot_general(a, b, (((1,), (0,)), ((), ())),
                           preferred_element_type=jnp.float32)


def _lin_body(h_ref, wl_ref, wr_ref, hw_ref, hr_ref):
    h = h_ref[...]
    hw_ref[...] = _dot(h, wl_ref[...])
    hr_ref[...] = _dot(h, wr_ref[...])


_tc_lin = pl.pallas_call(
    _lin_body,
    grid=(N // BM,),
    in_specs=[
        pl.BlockSpec((BM, D), lambda i: (i, 0)),
        pl.BlockSpec((D, D), lambda i: (0, 0)),
        pl.BlockSpec((D, D), lambda i: (0, 0)),
    ],
    out_specs=[
        pl.BlockSpec((BM, D), lambda i: (i, 0)),
        pl.BlockSpec((BM, D), lambda i: (i, 0)),
    ],
    out_shape=[
        jax.ShapeDtypeStruct((N, D), jnp.float32),
        jax.ShapeDtypeStruct((N, D), jnp.float32),
    ],
)


def _node_update(pa_ref, pb_ref, ca_ref, cb_ref, b_ref, hr_ref):
    agg = pa_ref[0] + pb_ref[0]
    cnt = ca_ref[0, :, 0:1] + cb_ref[0, :, 0:1]
    mean = agg / jnp.maximum(cnt, 1.0)
    return jnp.maximum(mean + b_ref[...] + hr_ref[...], 0.0)


def _comb_lin_body(pa_ref, pb_ref, ca_ref, cb_ref, b_ref, hr_ref,
                   wl_ref, wr_ref, hw_ref, hr2_ref):
    h = _node_update(pa_ref, pb_ref, ca_ref, cb_ref, b_ref, hr_ref)
    hw_ref[...] = _dot(h, wl_ref[...])
    hr2_ref[...] = _dot(h, wr_ref[...])


_tc_comb_lin = pl.pallas_call(
    _comb_lin_body,
    grid=(N // BM,),
    in_specs=[
        pl.BlockSpec((1, BM, D), lambda i: (0, i, 0)),
        pl.BlockSpec((1, BM, D), lambda i: (1, i, 0)),
        pl.BlockSpec((1, BM, D), lambda i: (0, i, 0)),
        pl.BlockSpec((1, BM, D), lambda i: (1, i, 0)),
        pl.BlockSpec((1, D), lambda i: (0, 0)),
        pl.BlockSpec((BM, D), lambda i: (i, 0)),
        pl.BlockSpec((D, D), lambda i: (0, 0)),
        pl.BlockSpec((D, D), lambda i: (0, 0)),
    ],
    out_specs=[
        pl.BlockSpec((BM, D), lambda i: (i, 0)),
        pl.BlockSpec((BM, D), lambda i: (i, 0)),
    ],
    out_shape=[
        jax.ShapeDtypeStruct((N, D), jnp.float32),
        jax.ShapeDtypeStruct((N, D), jnp.float32),
    ],
)


def _final_body(pa_ref, pb_ref, ca_ref, cb_ref, b_ref, hr_ref, out_ref):
    out_ref[...] = _node_update(pa_ref, pb_ref, ca_ref, cb_ref, b_ref, hr_ref)


_tc_final = pl.pallas_call(
    _final_body,
    grid=(N // BM,),
    in_specs=[
        pl.BlockSpec((1, BM, D), lambda i: (0, i, 0)),
        pl.BlockSpec((1, BM, D), lambda i: (1, i, 0)),
        pl.BlockSpec((1, BM, D), lambda i: (0, i, 0)),
        pl.BlockSpec((1, BM, D), lambda i: (1, i, 0)),
        pl.BlockSpec((1, D), lambda i: (0, 0)),
        pl.BlockSpec((BM, D), lambda i: (i, 0)),
    ],
    out_specs=pl.BlockSpec((BM, D), lambda i: (i, 0)),
    out_shape=jax.ShapeDtypeStruct((N, D), jnp.float32),
)


def kernel(x, edge_index, W_l1, b1, W_r1, W_l2, b2, W_r2):
    pk = (edge_index[0] | (edge_index[1] << 16)).reshape(NW, EPT)
    pad = jnp.full((NW, CH * K - EPT), PAD_DST << 16, jnp.int32)
    packed = jnp.concatenate([pk, pad], axis=1).reshape(NW, CH, K)

    e0 = jnp.zeros((K, D), jnp.float32).at[:, 0].set(1.0)
    cz = jnp.stack([jnp.zeros((K, D), jnp.float32), e0])
    cnt_parts = _sc_cnt(packed, cz)
    hw1, hr1 = _tc_lin(x, W_l1, W_r1)
    parts1 = _sc_agg(hw1, packed)
    b1r = b1.reshape(1, D)
    hw2, hr2 = _tc_comb_lin(parts1, parts1, cnt_parts, cnt_parts,
                            b1r, hr1, W_l2, W_r2)
    parts2 = _sc_agg(hw2, packed)
    return _tc_final(parts2, parts2, cnt_parts, cnt_parts,
                     b2.reshape(1, D), hr2)


# speedup vs baseline: 1.0116x; 1.0116x over previous
import functools

import jax
import jax.numpy as jnp
from jax import lax
from jax.experimental import pallas as pl
from jax.experimental.pallas import tpu as pltpu
from jax.experimental.pallas import tpu_sc as plsc

N = 10000
E = 320000
D = 128

NC = 2
NS = 16
NW = NC * NS
EPT = E // NW
K = 128
KG = 64
NBUF = 4
CHG = 160
CH = 80
N_PAD = 10240
RPT = N_PAD // NS
PAD_DST = N_PAD - 1

_mesh = functools.partial(
    plsc.VectorSubcoreMesh, core_axis_name="c", subcore_axis_name="s")


def _zero_rows(buf, rows, cols):
    zv = jnp.zeros((16,), jnp.float32)
    per_row = cols // 16

    def body(i, _):
        buf[i // per_row, pl.ds((i % per_row) * 16, 16)] = zv
        return 0

    lax.fori_loop(0, rows * per_row, body, 0, unroll=8)


def _fill_src(packed_v, ch, idx, k=K):
    for j in range(k // 16):
        w = packed_v[(ch * k + 16 * j) // K, pl.ds((ch * k + 16 * j) % K, 16)]
        idx[pl.ds(16 * j, 16)] = jnp.bitwise_and(w, 0xFFFF)


def _fill_dst(packed_v, ch, idx, k=K):
    for j in range(k // 16):
        w = packed_v[(ch * k + 16 * j) // K, pl.ds((ch * k + 16 * j) % K, 16)]
        idx[pl.ds(16 * j, 16)] = lax.shift_right_logical(w, 16)


@functools.partial(
    pl.kernel,
    out_type=jax.ShapeDtypeStruct((NC, N_PAD, D), jnp.float32),
    mesh=_mesh(),
    scratch_types=[
        pltpu.VMEM((CH, K), jnp.int32),
        pltpu.VMEM((K, D), jnp.float32),
        pltpu.VMEM((K,), jnp.int32),
        pltpu.SemaphoreType.DMA,
        pltpu.VMEM_SHARED((N_PAD, D), jnp.float32),
    ],
)
def _sc_cnt(edges_hbm, cz_hbm, out_hbm, packed_v, ones_v, di, semi, acc):
    cid = lax.axis_index("c")
    sid = lax.axis_index("s")
    wid = sid * NC + cid
    pltpu.async_copy(edges_hbm.at[wid], packed_v, semi)

    pltpu.sync_copy(cz_hbm.at[0], ones_v)
    base = sid * RPT
    for t in range(RPT // K):
        pltpu.sync_copy(ones_v, acc.at[pl.ds(base + t * K, K)])
    pltpu.sync_copy(cz_hbm.at[1], ones_v)
    pltpu.make_async_copy(edges_hbm.at[wid], packed_v, semi).wait()
    plsc.subcore_barrier()

    def body(ch, _):
        _fill_dst(packed_v, ch, di)
        pltpu.sync_copy(ones_v, acc.at[di], add=True)
        return 0

    lax.fori_loop(0, CH, body, 0)
    plsc.subcore_barrier()
    pltpu.sync_copy(acc.at[pl.ds(base, RPT)], out_hbm.at[cid, pl.ds(base, RPT)])


@functools.partial(
    pl.kernel,
    out_type=jax.ShapeDtypeStruct((NC, N_PAD, D), jnp.float32),
    mesh=_mesh(),
    scratch_types=[
        pltpu.VMEM((CH, K), jnp.int32),
        [pltpu.VMEM((KG, D), jnp.float32) for _ in range(NBUF)],
        [pltpu.VMEM((KG,), jnp.int32) for _ in range(NBUF)],
        pltpu.VMEM((KG,), jnp.int32),
        [pltpu.SemaphoreType.DMA for _ in range(NBUF)],
        pltpu.SemaphoreType.DMA,
        pltpu.VMEM_SHARED((N_PAD, D), jnp.float32),
    ],
)
def _sc_agg(table_hbm, edges_hbm, out_hbm,
            packed_v, bufs, sis, di, sems, semi, acc):
    cid = lax.axis_index("c")
    sid = lax.axis_index("s")
    wid = sid * NC + cid
    pltpu.async_copy(edges_hbm.at[wid], packed_v, semi)

    _zero_rows(bufs[0], KG, D)
    _zero_rows(bufs[1], KG, D)
    base = sid * RPT
    for t in range(RPT // K):
        pltpu.sync_copy(bufs[0], acc.at[pl.ds(base + t * K, KG)])
        pltpu.sync_copy(bufs[1], acc.at[pl.ds(base + t * K + KG, KG)])
    pltpu.make_async_copy(edges_hbm.at[wid], packed_v, semi).wait()
    plsc.subcore_barrier()

    def gather(si, buf, sem):
        pltpu.async_copy(table_hbm.at[si], buf, sem)

    def drain_g(si, buf, sem):
        pltpu.make_async_copy(table_hbm.at[si], buf, sem).wait()

    for b in range(NBUF):
        _fill_src(packed_v, b, sis[b], KG)
        gather(sis[b], bufs[b], sems[b])

    def body(i, _):
        for b in range(NBUF):
            ch = NBUF * i + b
            drain_g(sis[b], bufs[b], sems[b])
            _fill_dst(packed_v, ch, di, KG)
            pltpu.sync_copy(bufs[b], acc.at[di], add=True)

            @pl.when(ch + NBUF < CHG)
            def _():
                _fill_src(packed_v, ch + NBUF, sis[b], KG)
                gather(sis[b], bufs[b], sems[b])

        return 0

    lax.fori_loop(0, CHG // NBUF, body, 0)
    plsc.subcore_barrier()
    pltpu.sync_copy(acc.at[pl.ds(base, RPT)], out_hbm.at[cid, pl.ds(base, RPT)])


BM = 1000


def _dot(a, b):
    return lax.dot_general(a, b, (((1,), (0,)), ((), ())),
                           preferred_element_type=jnp.float32)


def _lin_body(h_ref, wl_ref, wr_ref, hw_ref, hr_ref):
    h = h_ref[...]
    hw_ref[...] = _dot(h, wl_ref[...])
    hr_ref[...] = _dot(h, wr_ref[...])


_tc_lin = pl.pallas_call(
    _lin_body,
    grid=(N // BM,),
    in_specs=[
        pl.BlockSpec((BM, D), lambda i: (i, 0)),
        pl.BlockSpec((D, D), lambda i: (0, 0)),
        pl.BlockSpec((D, D), lambda i: (0, 0)),
    ],
    out_specs=[
        pl.BlockSpec((BM, D), lambda i: (i, 0)),
        pl.BlockSpec((BM, D), lambda i: (i, 0)),
    ],
    out_shape=[
        jax.ShapeDtypeStruct((N, D), jnp.float32),
        jax.ShapeDtypeStruct((N, D), jnp.float32),
    ],
)


def _node_update(pa_ref, pb_ref, ca_ref, cb_ref, b_ref, hr_ref):
    agg = pa_ref[0] + pb_ref[0]
    cnt = ca_ref[0, :, 0:1] + cb_ref[0, :, 0:1]
    mean = agg / jnp.maximum(cnt, 1.0)
    return jnp.maximum(mean + b_ref[...] + hr_ref[...], 0.0)


def _comb_lin_body(pa_ref, pb_ref, ca_ref, cb_ref, b_ref, hr_ref,
                   wl_ref, wr_ref, hw_ref, hr2_ref):
    h = _node_update(pa_ref, pb_ref, ca_ref, cb_ref, b_ref, hr_ref)
    hw_ref[...] = _dot(h, wl_ref[...])
    hr2_ref[...] = _dot(h, wr_ref[...])


_tc_comb_lin = pl.pallas_call(
    _comb_lin_body,
    grid=(N // BM,),
    in_specs=[
        pl.BlockSpec((1, BM, D), lambda i: (0, i, 0)),
        pl.BlockSpec((1, BM, D), lambda i: (1, i, 0)),
        pl.BlockSpec((1, BM, D), lambda i: (0, i, 0)),
        pl.BlockSpec((1, BM, D), lambda i: (1, i, 0)),
        pl.BlockSpec((1, D), lambda i: (0, 0)),
        pl.BlockSpec((BM, D), lambda i: (i, 0)),
        pl.BlockSpec((D, D), lambda i: (0, 0)),
        pl.BlockSpec((D, D), lambda i: (0, 0)),
    ],
    out_specs=[
        pl.BlockSpec((BM, D), lambda i: (i, 0)),
        pl.BlockSpec((BM, D), lambda i: (i, 0)),
    ],
    out_shape=[
        jax.ShapeDtypeStruct((N, D), jnp.float32),
        jax.ShapeDtypeStruct((N, D), jnp.float32),
    ],
)


def _final_body(pa_ref, pb_ref, ca_ref, cb_ref, b_ref, hr_ref, out_ref):
    out_ref[...] = _node_update(pa_ref, pb_ref, ca_ref, cb_ref, b_ref, hr_ref)


_tc_final = pl.pallas_call(
    _final_body,
    grid=(N // BM,),
    in_specs=[
        pl.BlockSpec((1, BM, D), lambda i: (0, i, 0)),
        pl.BlockSpec((1, BM, D), lambda i: (1, i, 0)),
        pl.BlockSpec((1, BM, D), lambda i: (0, i, 0)),
        pl.BlockSpec((1, BM, D), lambda i: (1, i, 0)),
        pl.BlockSpec((1, D), lambda i: (0, 0)),
        pl.BlockSpec((BM, D), lambda i: (i, 0)),
    ],
    out_specs=pl.BlockSpec((BM, D), lambda i: (i, 0)),
    out_shape=jax.ShapeDtypeStruct((N, D), jnp.float32),
)


def kernel(x, edge_index, W_l1, b1, W_r1, W_l2, b2, W_r2):
    pk = (edge_index[0] | (edge_index[1] << 16)).reshape(NW, EPT)
    pad = jnp.full((NW, CH * K - EPT), PAD_DST << 16, jnp.int32)
    packed = jnp.concatenate([pk, pad], axis=1).reshape(NW, CH, K)

    e0 = jnp.zeros((K, D), jnp.float32).at[:, 0].set(1.0)
    cz = jnp.stack([jnp.zeros((K, D), jnp.float32), e0])
    cnt_parts = _sc_cnt(packed, cz)
    hw1, hr1 = _tc_lin(x, W_l1, W_r1)
    parts1 = _sc_agg(hw1, packed)
    b1r = b1.reshape(1, D)
    hw2, hr2 = _tc_comb_lin(parts1, parts1, cnt_parts, cnt_parts,
                            b1r, hr1, W_l2, W_r2)
    parts2 = _sc_agg(hw2, packed)
    return _tc_final(parts2, parts2, cnt_parts, cnt_parts,
                     b2.reshape(1, D), hr2)
